# baseline (speedup 1.0000x reference)
import jax
import jax.numpy as jnp
from jax import lax
from jax.experimental import pallas as pl
from jax.experimental.pallas import tpu as pltpu
from jax.experimental.pallas import tpu_sc as plsc

N_NODES = 10000
D = 128
NC = 2
NS = 16
NW = NC * NS
CHUNK = 128

E = 320000
NCHUNKS = -(-E // (NW * CHUNK))
NCHUNKS = NCHUNKS + (NCHUNKS % 2)
EPW = NCHUNKS * CHUNK
E_PAD = EPW * NW
HALF = NCHUNKS // 2

ZPT = 632
N_PAD = ZPT * NS
OPT = 624
OREM = N_NODES - OPT * NS


def _mm_body(x_ref, w_ref, b_ref, o_ref):
    o_ref[...] = (
        jnp.dot(x_ref[...], w_ref[...], preferred_element_type=jnp.float32)
        + b_ref[...]
    )


def _comb_body(p_ref, b_ref, o_ref):
    o_ref[...] = p_ref[0] + p_ref[1] + b_ref[...]


def _sc_body(h_hbm, src_hbm, dst_hbm, out_hbm,
             sall, dall, rows_a, rows_b, acc, sem):
    cid = lax.axis_index("c")
    sid = lax.axis_index("s")
    wid = cid * NS + sid

    z16 = jnp.zeros((16,), jnp.float32)

    def _zero_row(r, carry):
        for j in range(D // 16):
            rows_a[r, pl.ds(16 * j, 16)] = z16
        return carry

    lax.fori_loop(0, CHUNK, _zero_row, 0)

    zbase = pl.multiple_of(sid * ZPT, 8)
    for k in range(ZPT // CHUNK):
        pltpu.sync_copy(rows_a.at[pl.ds(0, CHUNK)],
                        acc.at[pl.ds(zbase + k * CHUNK, CHUNK)])
    zrem = ZPT % CHUNK
    if zrem:
        pltpu.sync_copy(rows_a.at[pl.ds(0, zrem)],
                        acc.at[pl.ds(zbase + (ZPT // CHUNK) * CHUNK, zrem)])

    plsc.subcore_barrier()

    for half in range(2):
        ibase = pl.multiple_of(wid * NCHUNKS + half * HALF, 8)
        pltpu.sync_copy(src_hbm.at[pl.ds(ibase, HALF)], sall)
        pltpu.sync_copy(dst_hbm.at[pl.ds(ibase, HALF)], dall)

        def _chunk(j, carry):
            pltpu.async_copy(h_hbm.at[sall.at[j]], rows_a, sem).wait()
            pltpu.sync_copy(rows_a, acc.at[dall.at[j]], add=True)
            return carry

        lax.fori_loop(0, HALF, _chunk, 0)

    plsc.subcore_barrier()

    obase = pl.multiple_of(sid * OPT, 8)
    for k in range(OPT // CHUNK):
        pltpu.sync_copy(acc.at[pl.ds(obase + k * CHUNK, CHUNK)],
                        out_hbm.at[cid].at[pl.ds(obase + k * CHUNK, CHUNK)])
    orem = OPT % CHUNK
    if orem:
        pltpu.sync_copy(acc.at[pl.ds(obase + (OPT // CHUNK) * CHUNK, orem)],
                        out_hbm.at[cid].at[pl.ds(obase + (OPT // CHUNK) * CHUNK, orem)])

    @pl.when(sid == NS - 1)
    def _tail():
        pltpu.sync_copy(acc.at[pl.ds(OPT * NS, OREM)],
                        out_hbm.at[cid].at[pl.ds(OPT * NS, OREM)])


def kernel(x, edge_index, W, b_dense, bias):
    src = edge_index[0].astype(jnp.int32)
    dst = edge_index[1].astype(jnp.int32)
    pad = E_PAD - E
    src = jnp.concatenate([src, jnp.zeros((pad,), jnp.int32)])
    dst = jnp.concatenate([dst, jnp.full((pad,), N_NODES, jnp.int32)])
    src2 = src.reshape(NW * NCHUNKS, CHUNK)
    dst2 = dst.reshape(NW * NCHUNKS, CHUNK)

    b2 = b_dense[None, :]
    h = pl.pallas_call(
        _mm_body,
        grid=(10,),
        in_specs=[
            pl.BlockSpec((N_NODES // 10, D), lambda i: (i, 0)),
            pl.BlockSpec((D, D), lambda i: (0, 0)),
            pl.BlockSpec((1, D), lambda i: (0, 0)),
        ],
        out_specs=pl.BlockSpec((N_NODES // 10, D), lambda i: (i, 0)),
        out_shape=jax.ShapeDtypeStruct((N_NODES, D), jnp.float32),
    )(x, W, b2)

    sc_fn = pl.kernel(
        _sc_body,
        out_type=jax.ShapeDtypeStruct((NC, N_NODES, D), jnp.float32),
        mesh=plsc.VectorSubcoreMesh(core_axis_name="c", subcore_axis_name="s"),
        scratch_types=[
            pltpu.VMEM((HALF, CHUNK), jnp.int32),
            pltpu.VMEM((HALF, CHUNK), jnp.int32),
            pltpu.VMEM((CHUNK, D), jnp.float32),
            pltpu.VMEM((CHUNK, D), jnp.float32),
            pltpu.VMEM_SHARED((N_PAD, D), jnp.float32),
            pltpu.SemaphoreType.DMA,
        ],
    )
    partials = sc_fn(h, src2, dst2)

    bias2 = bias[None, :]
    out = pl.pallas_call(
        _comb_body,
        grid=(10,),
        in_specs=[
            pl.BlockSpec((NC, N_NODES // 10, D), lambda i: (0, i, 0)),
            pl.BlockSpec((1, D), lambda i: (0, 0)),
        ],
        out_specs=pl.BlockSpec((N_NODES // 10, D), lambda i: (i, 0)),
        out_shape=jax.ShapeDtypeStruct((N_NODES, D), jnp.float32),
    )(partials, bias2)
    return out


# speedup vs baseline: 1.0534x; 1.0534x over previous
import jax
import jax.numpy as jnp
from jax import lax
from jax.experimental import pallas as pl
from jax.experimental.pallas import tpu as pltpu
from jax.experimental.pallas import tpu_sc as plsc

N_NODES = 10000
D = 128
NC = 2
NS = 16
NW = NC * NS
CHUNK = 128

E = 320000
NCHUNKS = -(-E // (NW * CHUNK))
NCHUNKS = NCHUNKS + (NCHUNKS % 2)
EPW = NCHUNKS * CHUNK
E_PAD = EPW * NW
GROUP = 8

ZPT = 632
N_PAD = ZPT * NS
OPT = 624
OREM = N_NODES - OPT * NS


def _mm_body(x_ref, w_ref, b_ref, o_ref):
    o_ref[...] = (
        jnp.dot(x_ref[...], w_ref[...], preferred_element_type=jnp.float32)
        + b_ref[...]
    )


def _comb_body(p_ref, b_ref, o_ref):
    o_ref[...] = p_ref[0] + p_ref[1] + b_ref[...]


def _sc_body(h_hbm, src_flat, dst_flat, out_hbm,
             srcv0, srcv1, dstv0, dstv1, rows_a, rows_b, acc, sem):
    cid = lax.axis_index("c")
    sid = lax.axis_index("s")
    wid = cid * NS + sid

    z16 = jnp.zeros((16,), jnp.float32)

    def _zero_row(r, carry):
        for j in range(D // 16):
            rows_a[r, pl.ds(16 * j, 16)] = z16
        return carry

    lax.fori_loop(0, CHUNK, _zero_row, 0)

    zbase = pl.multiple_of(sid * ZPT, 8)
    for k in range(ZPT // CHUNK):
        pltpu.sync_copy(rows_a.at[pl.ds(0, CHUNK)],
                        acc.at[pl.ds(zbase + k * CHUNK, CHUNK)])
    zrem = ZPT % CHUNK
    if zrem:
        pltpu.sync_copy(rows_a.at[pl.ds(0, zrem)],
                        acc.at[pl.ds(zbase + (ZPT // CHUNK) * CHUNK, zrem)])

    plsc.subcore_barrier()

    base0 = wid * EPW
    srcv = (srcv0, srcv1)
    dstv = (dstv0, dstv1)
    rows = (rows_a, rows_b)

    def _group(g, carry):
        gbase = pl.multiple_of(base0 + g * GROUP * CHUNK, CHUNK)
        pltpu.sync_copy(src_flat.at[pl.ds(gbase, CHUNK)], srcv0)
        pltpu.sync_copy(dst_flat.at[pl.ds(gbase, CHUNK)], dstv0)
        d = pltpu.async_copy(h_hbm.at[srcv0], rows_a, sem)
        for k in range(GROUP):
            p, q = k % 2, (k + 1) % 2
            if k + 1 < GROUP:
                nbase = pl.multiple_of(gbase + (k + 1) * CHUNK, CHUNK)
                pltpu.sync_copy(src_flat.at[pl.ds(nbase, CHUNK)], srcv[q])
                pltpu.sync_copy(dst_flat.at[pl.ds(nbase, CHUNK)], dstv[q])
            d.wait()
            if k + 1 < GROUP:
                d = pltpu.async_copy(h_hbm.at[srcv[q]], rows[q], sem)
            pltpu.sync_copy(rows[p], acc.at[dstv[p]], add=True)
        return carry

    lax.fori_loop(0, NCHUNKS // GROUP, _group, 0)

    plsc.subcore_barrier()

    obase = pl.multiple_of(sid * OPT, 8)
    for k in range(OPT // CHUNK):
        pltpu.sync_copy(acc.at[pl.ds(obase + k * CHUNK, CHUNK)],
                        out_hbm.at[cid].at[pl.ds(obase + k * CHUNK, CHUNK)])
    orem = OPT % CHUNK
    if orem:
        pltpu.sync_copy(acc.at[pl.ds(obase + (OPT // CHUNK) * CHUNK, orem)],
                        out_hbm.at[cid].at[pl.ds(obase + (OPT // CHUNK) * CHUNK, orem)])

    @pl.when(sid == NS - 1)
    def _tail():
        pltpu.sync_copy(acc.at[pl.ds(OPT * NS, OREM)],
                        out_hbm.at[cid].at[pl.ds(OPT * NS, OREM)])


def kernel(x, edge_index, W, b_dense, bias):
    src = edge_index[0].astype(jnp.int32)
    dst = edge_index[1].astype(jnp.int32)
    pad = E_PAD - E
    src = jnp.concatenate([src, jnp.zeros((pad,), jnp.int32)])
    dst = jnp.concatenate([dst, jnp.full((pad,), N_NODES, jnp.int32)])

    b2 = b_dense[None, :]
    h = pl.pallas_call(
        _mm_body,
        grid=(10,),
        in_specs=[
            pl.BlockSpec((N_NODES // 10, D), lambda i: (i, 0)),
            pl.BlockSpec((D, D), lambda i: (0, 0)),
            pl.BlockSpec((1, D), lambda i: (0, 0)),
        ],
        out_specs=pl.BlockSpec((N_NODES // 10, D), lambda i: (i, 0)),
        out_shape=jax.ShapeDtypeStruct((N_NODES, D), jnp.float32),
    )(x, W, b2)

    sc_fn = pl.kernel(
        _sc_body,
        out_type=jax.ShapeDtypeStruct((NC, N_NODES, D), jnp.float32),
        mesh=plsc.VectorSubcoreMesh(core_axis_name="c", subcore_axis_name="s"),
        scratch_types=[
            pltpu.VMEM((CHUNK,), jnp.int32),
            pltpu.VMEM((CHUNK,), jnp.int32),
            pltpu.VMEM((CHUNK,), jnp.int32),
            pltpu.VMEM((CHUNK,), jnp.int32),
            pltpu.VMEM((CHUNK, D), jnp.float32),
            pltpu.VMEM((CHUNK, D), jnp.float32),
            pltpu.VMEM_SHARED((N_PAD, D), jnp.float32),
            pltpu.SemaphoreType.DMA,
        ],
    )
    partials = sc_fn(h, src, dst)

    bias2 = bias[None, :]
    out = pl.pallas_call(
        _comb_body,
        grid=(10,),
        in_specs=[
            pl.BlockSpec((NC, N_NODES // 10, D), lambda i: (0, i, 0)),
            pl.BlockSpec((1, D), lambda i: (0, 0)),
        ],
        out_specs=pl.BlockSpec((N_NODES // 10, D), lambda i: (i, 0)),
        out_shape=jax.ShapeDtypeStruct((N_NODES, D), jnp.float32),
    )(partials, bias2)
    return out
